# Initial kernel scaffold; baseline (speedup 1.0000x reference)
import jax
import jax.numpy as jnp
from jax.experimental import pallas as pl


def kernel(x, gate_w, gate_b, w1, b1, w2, b2):
    raise NotImplementedError("write your pallas kernel here")


# speedup vs baseline: 2.9580x; 2.9580x over previous
import functools

import jax
import jax.numpy as jnp
from jax.experimental import pallas as pl
from jax.experimental.pallas import tpu as pltpu

D = 768
E = 8

_INV_SQRT2 = 0.7071067811865476


def _gelu_exact(h):
    return 0.5 * h * (1.0 + jax.lax.erf(h * _INV_SQRT2))


def _fused_moe_body(x_ref, gw_ref, gb_ref, w1_ref, b1_ref, w2_ref, b2_ref,
                    out_ref, assign_ref, wmax_ref):
    e = pl.program_id(0)

    @pl.when(e == 0)
    def _gating():
        x = x_ref[...]
        logits = jnp.dot(x, gw_ref[...], preferred_element_type=jnp.float32)
        logits = logits + gb_ref[...]
        m = jnp.max(logits, axis=-1, keepdims=True)
        ex = jnp.exp(logits - m)
        probs = ex / jnp.sum(ex, axis=-1, keepdims=True)
        assign_ref[...] = jnp.argmax(probs, axis=-1)[:, None]
        wmax_ref[...] = jnp.max(probs, axis=-1, keepdims=True)
        out_ref[...] = jnp.zeros_like(out_ref)

    x = x_ref[...]
    h = jnp.dot(x, w1_ref[0], preferred_element_type=jnp.float32) + b1_ref[0]
    h = _gelu_exact(h)
    y = jnp.dot(h, w2_ref[0], preferred_element_type=jnp.float32) + b2_ref[0]
    sel = jnp.where(assign_ref[...] == e, wmax_ref[...], 0.0)
    out_ref[...] += sel * y


def kernel(x, gate_w, gate_b, w1, b1, w2, b2):
    n, t, d = x.shape
    nt = n * t
    x_flat = x.reshape(nt, d)

    out = pl.pallas_call(
        _fused_moe_body,
        grid=(E,),
        in_specs=[
            pl.BlockSpec((nt, d), lambda e: (0, 0)),
            pl.BlockSpec((d, E), lambda e: (0, 0)),
            pl.BlockSpec((1, E), lambda e: (0, 0)),
            pl.BlockSpec((1, d, d), lambda e: (e, 0, 0)),
            pl.BlockSpec((1, 1, d), lambda e: (e, 0, 0)),
            pl.BlockSpec((1, d, d), lambda e: (e, 0, 0)),
            pl.BlockSpec((1, 1, d), lambda e: (e, 0, 0)),
        ],
        out_specs=pl.BlockSpec((nt, d), lambda e: (0, 0)),
        out_shape=jax.ShapeDtypeStruct((nt, d), jnp.float32),
        scratch_shapes=[pltpu.VMEM((nt, 1), jnp.int32),
                        pltpu.VMEM((nt, 1), jnp.float32)],
        compiler_params=pltpu.CompilerParams(
            dimension_semantics=("arbitrary",),
        ),
    )(x_flat, gate_w, gate_b.reshape(1, E),
      w1, b1.reshape(E, 1, d), w2, b2.reshape(E, 1, d))
    return out.reshape(n, t, d)
